# baseline (speedup 1.0000x reference)
import jax
import jax.numpy as jnp
from jax.experimental import pallas as pl
from jax.experimental.pallas import tpu as pltpu

B = 4
S = 2048
D = 768
P = 1000
TOP_K = 4
BLK = 256
N_BLK = S // BLK
OUT_DEPTH = 3


def _in_copy(x_hbm, xbuf, in_sems, blk_idx):
    return pltpu.make_async_copy(
        x_hbm.at[:, pl.ds(pl.multiple_of(blk_idx * BLK, BLK), BLK), :],
        xbuf.at[blk_idx],
        in_sems.at[blk_idx])


def _out_copy(sbuf, out_hbm, out_sems, blk_idx):
    slot = jax.lax.rem(blk_idx, OUT_DEPTH)
    return pltpu.make_async_copy(
        sbuf.at[slot],
        out_hbm.at[:, pl.ds(pl.multiple_of(blk_idx * BLK, BLK), BLK), :],
        out_sems.at[slot])


def _body(x_hbm, prompt_hbm, out_hbm, sim_ref, idx_ref, rsim_ref,
          xbuf, sbuf, pbuf, acc, tail, first4, head,
          in_sems, out_sems, p_sem, head_sem, tail_sem):
    i = pl.program_id(0)
    slot = jax.lax.rem(i, OUT_DEPTH)

    @pl.when(i == 0)
    def _():
        acc[...] = jnp.zeros_like(acc)
        tail[...] = jnp.zeros_like(tail)
        _in_copy(x_hbm, xbuf, in_sems, 0).start()

    _in_copy(x_hbm, xbuf, in_sems, i).wait()

    @pl.when(i == 0)
    def _():
        for b in range(1, N_BLK):
            _in_copy(x_hbm, xbuf, in_sems, b).start()
        pltpu.make_async_copy(prompt_hbm, pbuf, p_sem).start()

    v = xbuf[i]
    acc[...] += jnp.sum(v, axis=1)
    shifted = jnp.concatenate([tail[...], v[:, :BLK - TOP_K, :]], axis=1)
    tail[...] = v[:, BLK - TOP_K:, :]

    @pl.when(i == 0)
    def _():
        first4[...] = v[:, :TOP_K, :]

    @pl.when(i >= OUT_DEPTH)
    def _():
        _out_copy(sbuf, out_hbm, out_sems, i - OUT_DEPTH).wait()

    sbuf[slot] = shifted
    _out_copy(sbuf, out_hbm, out_sems, i).start()

    @pl.when(i == N_BLK - 1)
    def _():
        mean = acc[...] * (1.0 / S)
        xn = mean * jax.lax.rsqrt(
            jnp.maximum(jnp.sum(mean * mean, axis=1, keepdims=True), 1e-12))
        pltpu.make_async_copy(prompt_hbm, pbuf, p_sem).wait()
        p = pbuf[...]
        pn = p * jax.lax.rsqrt(
            jnp.maximum(jnp.sum(p * p, axis=1, keepdims=True), 1e-12))
        sim = jax.lax.dot_general(
            xn, pn, (((1,), (1,)), ((), ())),
            preferred_element_type=jnp.float32)
        sim_ref[...] = sim

        iota = jax.lax.broadcasted_iota(jnp.int32, (B, P), 1)
        s = sim
        total = jnp.float32(0.0)
        idx_cols = []
        bp_cols = []
        for k in range(TOP_K):
            m = jnp.max(s, axis=1, keepdims=True)
            eq = s == m
            ik = jnp.min(jnp.where(eq, iota, P), axis=1)
            sel = iota == ik[:, None]
            idx_cols.append(ik)
            total += jnp.sum(m)
            bp_cols.append(jax.lax.dot_general(
                sel.astype(jnp.float32), p, (((1,), (0,)), ((), ())),
                preferred_element_type=jnp.float32))
            s = jnp.where(sel, -jnp.inf, s)
        idx_ref[...] = jnp.stack(idx_cols, axis=1)
        rsim_ref[...] = jnp.reshape(total * (1.0 / B), (1, 1))

        head[...] = jnp.concatenate(
            [jnp.stack(bp_cols, axis=1), first4[...]], axis=1)
        hcopy = pltpu.make_async_copy(
            head, out_hbm.at[:, pl.ds(0, 2 * TOP_K), :], head_sem)
        hcopy.start()
        tcopy = pltpu.make_async_copy(
            tail, out_hbm.at[:, pl.ds(S, TOP_K), :], tail_sem)
        tcopy.start()
        for b in range(OUT_DEPTH - 1, 0, -1):
            _out_copy(sbuf, out_hbm, out_sems, i - b).wait()
        _out_copy(sbuf, out_hbm, out_sems, i).wait()
        hcopy.wait()
        tcopy.wait()


def kernel(x_embed, prompt):
    out_shapes = (
        jax.ShapeDtypeStruct((B, TOP_K + S, D), jnp.float32),
        jax.ShapeDtypeStruct((B, P), jnp.float32),
        jax.ShapeDtypeStruct((B, TOP_K), jnp.int32),
        jax.ShapeDtypeStruct((1, 1), jnp.float32),
    )
    prompted, sim, idx, rsim = pl.pallas_call(
        _body,
        grid=(N_BLK,),
        in_specs=[
            pl.BlockSpec(memory_space=pl.MemorySpace.ANY),
            pl.BlockSpec(memory_space=pl.MemorySpace.ANY),
        ],
        out_specs=(
            pl.BlockSpec(memory_space=pl.MemorySpace.ANY),
            pl.BlockSpec((B, P), lambda i: (0, 0)),
            pl.BlockSpec((B, TOP_K), lambda i: (0, 0)),
            pl.BlockSpec((1, 1), lambda i: (0, 0)),
        ),
        out_shape=out_shapes,
        scratch_shapes=[
            pltpu.VMEM((N_BLK, B, BLK, D), jnp.float32),
            pltpu.VMEM((OUT_DEPTH, B, BLK, D), jnp.float32),
            pltpu.VMEM((P, D), jnp.float32),
            pltpu.VMEM((B, D), jnp.float32),
            pltpu.VMEM((B, TOP_K, D), jnp.float32),
            pltpu.VMEM((B, TOP_K, D), jnp.float32),
            pltpu.VMEM((B, 2 * TOP_K, D), jnp.float32),
            pltpu.SemaphoreType.DMA((N_BLK,)),
            pltpu.SemaphoreType.DMA((OUT_DEPTH,)),
            pltpu.SemaphoreType.DMA,
            pltpu.SemaphoreType.DMA,
            pltpu.SemaphoreType.DMA,
        ],
        compiler_params=pltpu.CompilerParams(
            dimension_semantics=("arbitrary",),
        ),
    )(x_embed, prompt)
    return prompted, rsim[0, 0], sim, idx


# speedup vs baseline: 1.0261x; 1.0261x over previous
import jax
import jax.numpy as jnp
from jax.experimental import pallas as pl
from jax.experimental.pallas import tpu as pltpu

B = 4
S = 2048
D = 768
P = 1000
TOP_K = 4
BLK = 256
N_BLK = S // BLK
OUT_DEPTH = 4


def _in_copy(x_hbm, xbuf, in_sems, blk_idx):
    return pltpu.make_async_copy(
        x_hbm.at[:, pl.ds(pl.multiple_of(blk_idx * BLK, BLK), BLK), :],
        xbuf.at[blk_idx],
        in_sems.at[blk_idx])


def _out_copy(sbuf, out_hbm, out_sems, blk_idx):
    slot = jax.lax.rem(blk_idx, OUT_DEPTH)
    return pltpu.make_async_copy(
        sbuf.at[slot],
        out_hbm.at[:, pl.ds(pl.multiple_of(blk_idx * BLK, BLK), BLK), :],
        out_sems.at[slot])


def _body(x_hbm, prompt_hbm, out_hbm, sim_ref, idx_ref, rsim_ref,
          xbuf, sbuf, pbuf, acc, tail, first4, head,
          in_sems, out_sems, p_sem, head_sem, tail_sem):
    i = pl.program_id(0)
    slot = jax.lax.rem(i, OUT_DEPTH)

    @pl.when(i == 0)
    def _():
        acc[...] = jnp.zeros_like(acc)
        tail[...] = jnp.zeros_like(tail)
        for b in range(N_BLK):
            _in_copy(x_hbm, xbuf, in_sems, b).start()
        pltpu.make_async_copy(prompt_hbm, pbuf, p_sem).start()

    _in_copy(x_hbm, xbuf, in_sems, i).wait()
    v = xbuf[i]
    acc[...] += jnp.sum(v, axis=1)
    shifted = jnp.concatenate([tail[...], v[:, :BLK - TOP_K, :]], axis=1)
    tail[...] = v[:, BLK - TOP_K:, :]

    @pl.when(i == 0)
    def _():
        first4[...] = v[:, :TOP_K, :]

    @pl.when(i >= OUT_DEPTH)
    def _():
        _out_copy(sbuf, out_hbm, out_sems, i - OUT_DEPTH).wait()

    sbuf[slot] = shifted
    _out_copy(sbuf, out_hbm, out_sems, i).start()

    @pl.when(i == N_BLK - 1)
    def _():
        mean = acc[...] * (1.0 / S)
        xn = mean * jax.lax.rsqrt(
            jnp.maximum(jnp.sum(mean * mean, axis=1, keepdims=True), 1e-12))
        pltpu.make_async_copy(prompt_hbm, pbuf, p_sem).wait()
        p = pbuf[...]
        pn = p * jax.lax.rsqrt(
            jnp.maximum(jnp.sum(p * p, axis=1, keepdims=True), 1e-12))
        sim = jax.lax.dot_general(
            xn, pn, (((1,), (1,)), ((), ())),
            preferred_element_type=jnp.float32)
        sim_ref[...] = sim

        iota = jax.lax.broadcasted_iota(jnp.int32, (B, P), 1)
        s = sim
        total = jnp.float32(0.0)
        idx_cols = []
        bp_cols = []
        for k in range(TOP_K):
            m = jnp.max(s, axis=1, keepdims=True)
            eq = s == m
            ik = jnp.min(jnp.where(eq, iota, P), axis=1)
            sel = iota == ik[:, None]
            idx_cols.append(ik)
            total += jnp.sum(m)
            bp_cols.append(jax.lax.dot_general(
                sel.astype(jnp.float32), p, (((1,), (0,)), ((), ())),
                preferred_element_type=jnp.float32))
            s = jnp.where(sel, -jnp.inf, s)
        idx_ref[...] = jnp.stack(idx_cols, axis=1)
        rsim_ref[...] = jnp.reshape(total * (1.0 / B), (1, 1))

        head[...] = jnp.concatenate(
            [jnp.stack(bp_cols, axis=1), first4[...]], axis=1)
        hcopy = pltpu.make_async_copy(
            head, out_hbm.at[:, pl.ds(0, 2 * TOP_K), :], head_sem)
        hcopy.start()
        tcopy = pltpu.make_async_copy(
            tail, out_hbm.at[:, pl.ds(S, TOP_K), :], tail_sem)
        tcopy.start()
        for b in range(OUT_DEPTH - 1, 0, -1):
            _out_copy(sbuf, out_hbm, out_sems, i - b).wait()
        _out_copy(sbuf, out_hbm, out_sems, i).wait()
        hcopy.wait()
        tcopy.wait()


def kernel(x_embed, prompt):
    out_shapes = (
        jax.ShapeDtypeStruct((B, TOP_K + S, D), jnp.float32),
        jax.ShapeDtypeStruct((B, P), jnp.float32),
        jax.ShapeDtypeStruct((B, TOP_K), jnp.int32),
        jax.ShapeDtypeStruct((1, 1), jnp.float32),
    )
    prompted, sim, idx, rsim = pl.pallas_call(
        _body,
        grid=(N_BLK,),
        in_specs=[
            pl.BlockSpec(memory_space=pl.MemorySpace.ANY),
            pl.BlockSpec(memory_space=pl.MemorySpace.ANY),
        ],
        out_specs=(
            pl.BlockSpec(memory_space=pl.MemorySpace.ANY),
            pl.BlockSpec((B, P), lambda i: (0, 0)),
            pl.BlockSpec((B, TOP_K), lambda i: (0, 0)),
            pl.BlockSpec((1, 1), lambda i: (0, 0)),
        ),
        out_shape=out_shapes,
        scratch_shapes=[
            pltpu.VMEM((N_BLK, B, BLK, D), jnp.float32),
            pltpu.VMEM((OUT_DEPTH, B, BLK, D), jnp.float32),
            pltpu.VMEM((P, D), jnp.float32),
            pltpu.VMEM((B, D), jnp.float32),
            pltpu.VMEM((B, TOP_K, D), jnp.float32),
            pltpu.VMEM((B, TOP_K, D), jnp.float32),
            pltpu.VMEM((B, 2 * TOP_K, D), jnp.float32),
            pltpu.SemaphoreType.DMA((N_BLK,)),
            pltpu.SemaphoreType.DMA((OUT_DEPTH,)),
            pltpu.SemaphoreType.DMA,
            pltpu.SemaphoreType.DMA,
            pltpu.SemaphoreType.DMA,
        ],
        compiler_params=pltpu.CompilerParams(
            dimension_semantics=("arbitrary",),
        ),
    )(x_embed, prompt)
    return prompted, rsim[0, 0], sim, idx


# speedup vs baseline: 1.0319x; 1.0057x over previous
import jax
import jax.numpy as jnp
from jax.experimental import pallas as pl
from jax.experimental.pallas import tpu as pltpu

B = 4
S = 2048
D = 768
P = 1000
TOP_K = 4
BLK = 256
N_BLK = S // BLK
OUT_DEPTH = 6


def _in_copy(x_hbm, xbuf, in_sems, blk_idx):
    return pltpu.make_async_copy(
        x_hbm.at[:, pl.ds(pl.multiple_of(blk_idx * BLK, BLK), BLK), :],
        xbuf.at[blk_idx],
        in_sems.at[blk_idx])


def _out_copy(sbuf, out_hbm, out_sems, blk_idx):
    slot = jax.lax.rem(blk_idx, OUT_DEPTH)
    return pltpu.make_async_copy(
        sbuf.at[slot],
        out_hbm.at[:, pl.ds(pl.multiple_of(blk_idx * BLK, BLK), BLK), :],
        out_sems.at[slot])


def _body(x_hbm, prompt_hbm, out_hbm, sim_ref, idx_ref, rsim_ref,
          xbuf, sbuf, pbuf, acc, tail, first4, head,
          in_sems, out_sems, p_sem, head_sem, tail_sem):
    i = pl.program_id(0)
    slot = jax.lax.rem(i, OUT_DEPTH)

    @pl.when(i == 0)
    def _():
        acc[...] = jnp.zeros_like(acc)
        tail[...] = jnp.zeros_like(tail)
        for b in range(N_BLK):
            _in_copy(x_hbm, xbuf, in_sems, b).start()
        pltpu.make_async_copy(prompt_hbm, pbuf, p_sem).start()

    _in_copy(x_hbm, xbuf, in_sems, i).wait()
    v = xbuf[i]
    acc[...] += jnp.sum(v, axis=1)
    shifted = jnp.concatenate([tail[...], v[:, :BLK - TOP_K, :]], axis=1)
    tail[...] = v[:, BLK - TOP_K:, :]

    @pl.when(i == 0)
    def _():
        first4[...] = v[:, :TOP_K, :]

    @pl.when(i >= OUT_DEPTH)
    def _():
        _out_copy(sbuf, out_hbm, out_sems, i - OUT_DEPTH).wait()

    sbuf[slot] = shifted
    _out_copy(sbuf, out_hbm, out_sems, i).start()

    @pl.when(i == N_BLK - 1)
    def _():
        mean = acc[...] * (1.0 / S)
        xn = mean * jax.lax.rsqrt(
            jnp.maximum(jnp.sum(mean * mean, axis=1, keepdims=True), 1e-12))
        pltpu.make_async_copy(prompt_hbm, pbuf, p_sem).wait()
        p = pbuf[...]
        pn = p * jax.lax.rsqrt(
            jnp.maximum(jnp.sum(p * p, axis=1, keepdims=True), 1e-12))
        sim = jax.lax.dot_general(
            xn, pn, (((1,), (1,)), ((), ())),
            preferred_element_type=jnp.float32)
        sim_ref[...] = sim

        iota = jax.lax.broadcasted_iota(jnp.int32, (B, P), 1)
        s = sim
        total = jnp.float32(0.0)
        idx_cols = []
        bp_cols = []
        for k in range(TOP_K):
            m = jnp.max(s, axis=1, keepdims=True)
            eq = s == m
            ik = jnp.min(jnp.where(eq, iota, P), axis=1)
            sel = iota == ik[:, None]
            idx_cols.append(ik)
            total += jnp.sum(m)
            bp_cols.append(jax.lax.dot_general(
                sel.astype(jnp.float32), p, (((1,), (0,)), ((), ())),
                preferred_element_type=jnp.float32))
            s = jnp.where(sel, -jnp.inf, s)
        idx_ref[...] = jnp.stack(idx_cols, axis=1)
        rsim_ref[...] = jnp.reshape(total * (1.0 / B), (1, 1))

        head[...] = jnp.concatenate(
            [jnp.stack(bp_cols, axis=1), first4[...]], axis=1)
        hcopy = pltpu.make_async_copy(
            head, out_hbm.at[:, pl.ds(0, 2 * TOP_K), :], head_sem)
        hcopy.start()
        tcopy = pltpu.make_async_copy(
            tail, out_hbm.at[:, pl.ds(S, TOP_K), :], tail_sem)
        tcopy.start()
        for b in range(OUT_DEPTH - 1, 0, -1):
            _out_copy(sbuf, out_hbm, out_sems, i - b).wait()
        _out_copy(sbuf, out_hbm, out_sems, i).wait()
        hcopy.wait()
        tcopy.wait()


def kernel(x_embed, prompt):
    out_shapes = (
        jax.ShapeDtypeStruct((B, TOP_K + S, D), jnp.float32),
        jax.ShapeDtypeStruct((B, P), jnp.float32),
        jax.ShapeDtypeStruct((B, TOP_K), jnp.int32),
        jax.ShapeDtypeStruct((1, 1), jnp.float32),
    )
    prompted, sim, idx, rsim = pl.pallas_call(
        _body,
        grid=(N_BLK,),
        in_specs=[
            pl.BlockSpec(memory_space=pl.MemorySpace.ANY),
            pl.BlockSpec(memory_space=pl.MemorySpace.ANY),
        ],
        out_specs=(
            pl.BlockSpec(memory_space=pl.MemorySpace.ANY),
            pl.BlockSpec((B, P), lambda i: (0, 0)),
            pl.BlockSpec((B, TOP_K), lambda i: (0, 0)),
            pl.BlockSpec((1, 1), lambda i: (0, 0)),
        ),
        out_shape=out_shapes,
        scratch_shapes=[
            pltpu.VMEM((N_BLK, B, BLK, D), jnp.float32),
            pltpu.VMEM((OUT_DEPTH, B, BLK, D), jnp.float32),
            pltpu.VMEM((P, D), jnp.float32),
            pltpu.VMEM((B, D), jnp.float32),
            pltpu.VMEM((B, TOP_K, D), jnp.float32),
            pltpu.VMEM((B, TOP_K, D), jnp.float32),
            pltpu.VMEM((B, 2 * TOP_K, D), jnp.float32),
            pltpu.SemaphoreType.DMA((N_BLK,)),
            pltpu.SemaphoreType.DMA((OUT_DEPTH,)),
            pltpu.SemaphoreType.DMA,
            pltpu.SemaphoreType.DMA,
            pltpu.SemaphoreType.DMA,
        ],
        compiler_params=pltpu.CompilerParams(
            dimension_semantics=("arbitrary",),
        ),
    )(x_embed, prompt)
    return prompted, rsim[0, 0], sim, idx
